# Initial kernel scaffold; baseline (speedup 1.0000x reference)
import jax
import jax.numpy as jnp
from jax.experimental import pallas as pl


def kernel(x, edge_index, edge_attr, y, W_edge, b_edge, W, b):
    raise NotImplementedError("write your pallas kernel here")


# speedup vs baseline: 2.2621x; 2.2621x over previous
import functools

import jax
import jax.numpy as jnp
from jax import lax
from jax.experimental import pallas as pl
from jax.experimental.pallas import tpu as pltpu
from jax.experimental.pallas import tpu_sc as plsc


def _edge_lin_body(a_ref, w_ref, b_ref, o_ref):
    o_ref[...] = (
        jnp.dot(a_ref[...], w_ref[...], preferred_element_type=jnp.float32)
        + b_ref[...]
    )


def _edge_linear(edge_attr, W_edge, b_edge, block_e):
    E, DE = edge_attr.shape
    D = W_edge.shape[1]
    grid = E // block_e
    return pl.pallas_call(
        _edge_lin_body,
        grid=(grid,),
        in_specs=[
            pl.BlockSpec((block_e, DE), lambda i: (i, 0)),
            pl.BlockSpec((DE, D), lambda i: (0, 0)),
            pl.BlockSpec((1, D), lambda i: (0, 0)),
        ],
        out_specs=pl.BlockSpec((block_e, D), lambda i: (i, 0)),
        out_shape=jax.ShapeDtypeStruct((E, D), jnp.float32),
    )(edge_attr, W_edge, b_edge.reshape(1, D))


def _sc_aggregate(x, src, dst, e, zeros):
    N, D = x.shape
    NPAD = zeros.shape[0]
    E = src.shape[0]
    info = plsc.get_sparse_core_info()
    NC, NS = info.num_cores, info.num_subcores
    NW = NC * NS
    EPT = E // NW
    CB = 80
    NCHUNK = EPT // CB
    assert EPT % CB == 0 and E % NW == 0 and NPAD % (8 * NS) == 0
    RPT = NPAD // NS

    mesh = plsc.VectorSubcoreMesh(core_axis_name="c", subcore_axis_name="s")

    @functools.partial(
        pl.kernel,
        out_type=jax.ShapeDtypeStruct((NC, NPAD, D), jnp.float32),
        mesh=mesh,
        scratch_types=[
            pltpu.VMEM((CB,), jnp.int32),
            pltpu.VMEM((CB,), jnp.int32),
            pltpu.VMEM((CB, D), jnp.float32),
            pltpu.VMEM((CB, D), jnp.float32),
            pltpu.VMEM_SHARED((NPAD, D), jnp.float32),
            pltpu.SemaphoreType.DMA,
        ],
    )
    def body(x_hbm, src_hbm, dst_hbm, e_hbm, zero_hbm, out_hbm,
             src_v, dst_v, xbuf, ebuf, acc, gsem):
        c = lax.axis_index("c")
        s = lax.axis_index("s")
        wid = c * NS + s

        pltpu.sync_copy(zero_hbm.at[pl.ds(s * RPT, RPT)],
                        acc.at[pl.ds(s * RPT, RPT)])
        plsc.subcore_barrier()

        base0 = wid * EPT

        def chunk(i, carry):
            base = pl.multiple_of(base0 + i * CB, 8)
            pltpu.sync_copy(src_hbm.at[pl.ds(base, CB)], src_v)
            pltpu.sync_copy(dst_hbm.at[pl.ds(base, CB)], dst_v)
            pltpu.sync_copy(e_hbm.at[pl.ds(base, CB)], ebuf)
            pltpu.async_copy(x_hbm.at[src_v], xbuf, gsem).wait()

            def row(r, carry2):
                for j in range(D // 16):
                    sl = pl.ds(j * 16, 16)
                    ebuf[r, sl] = jnp.maximum(xbuf[r, sl] + ebuf[r, sl], 0.0)
                return carry2

            lax.fori_loop(0, CB, row, 0)
            pltpu.sync_copy(ebuf, acc.at[dst_v], add=True)
            return carry

        lax.fori_loop(0, NCHUNK, chunk, 0)
        plsc.subcore_barrier()
        pltpu.sync_copy(acc.at[pl.ds(s * RPT, RPT)],
                        out_hbm.at[c, pl.ds(s * RPT, RPT)])

    return body(x, src, dst, e, zeros)


def _final_body(x_ref, p_ref, w_ref, b_ref, o_ref):
    h = x_ref[...] + p_ref[0] + p_ref[1]
    o_ref[...] = (
        jnp.dot(h, w_ref[...], preferred_element_type=jnp.float32)
        + b_ref[...]
    )


def _final(x, parts, W, b, block_n):
    N, D = x.shape
    grid = N // block_n
    return pl.pallas_call(
        _final_body,
        grid=(grid,),
        in_specs=[
            pl.BlockSpec((block_n, D), lambda i: (i, 0)),
            pl.BlockSpec((2, block_n, D), lambda i: (0, i, 0)),
            pl.BlockSpec((D, D), lambda i: (0, 0)),
            pl.BlockSpec((1, D), lambda i: (0, 0)),
        ],
        out_specs=pl.BlockSpec((block_n, D), lambda i: (i, 0)),
        out_shape=jax.ShapeDtypeStruct((N, D), jnp.float32),
    )(x, parts, W, b.reshape(1, D))


def kernel(x, edge_index, edge_attr, y, W_edge, b_edge, W, b):
    N, D = x.shape
    src = edge_index[0]
    dst = edge_index[1]
    e = _edge_linear(edge_attr, W_edge, b_edge, block_e=1280)
    npad = -(-N // 128) * 128
    zeros = jnp.zeros((npad, D), jnp.float32)
    parts = _sc_aggregate(x, src, dst, e, zeros)
    pred = _final(x, parts, W, b, block_n=1000)
    return (pred, y)
